# baseline (speedup 1.0000x reference)
import functools

import jax
import jax.numpy as jnp
from jax import lax
from jax.experimental import pallas as pl
from jax.experimental.pallas import tpu as pltpu
from jax.experimental.pallas import tpu_sc as plsc

_EMB = 128
_NC = 2
_NS = 16
_NW = _NC * _NS
_L = 16
_BLK = 128

_CHUNK = 64
_UNROLL = 8


def _vsqrt(x):
    i = plsc.bitcast(x, jnp.int32)
    y = plsc.bitcast(jnp.int32(0x5F3759DF) - (i >> 1), jnp.float32)
    for _ in range(3):
        y = y * (1.5 - 0.5 * x * y * y)
    return x * y


@functools.partial(jax.jit, static_argnums=(2,))
def _sc_forward(idx3, table, batch):
    pairs_per_w = batch // _NW
    blocks_per_w = pairs_per_w // _BLK
    nchunk = pairs_per_w // _CHUNK
    groups = _CHUNK // _L
    mesh = plsc.VectorSubcoreMesh(core_axis_name="c", subcore_axis_name="s")

    def body(idx_hbm, table_hbm, out_hbm, idx_v, bc, bd, out_v,
             sc0, sd0, sc1, sd1):
        wid = lax.axis_index("s") * _NC + lax.axis_index("c")
        pltpu.sync_copy(idx_hbm.at[pl.ds(wid * blocks_per_w, blocks_per_w)],
                        idx_v)

        def start(ch, half):
            blk = ch // 2
            semc, semd = (sc0, sd0) if half == 0 else (sc1, sd1)
            dst = pl.ds(half * _CHUNK, _CHUNK)
            pltpu.async_copy(
                table_hbm.at[idx_v.at[blk, 0, pl.ds(half * _CHUNK, _CHUNK)]],
                bc.at[dst], semc)
            pltpu.async_copy(
                table_hbm.at[idx_v.at[blk, 1, pl.ds(half * _CHUNK, _CHUNK)]],
                bd.at[dst], semd)

        def wait(half):
            semc, semd = (sc0, sd0) if half == 0 else (sc1, sd1)
            dst = pl.ds(half * _CHUNK, _CHUNK)
            pltpu.make_async_copy(
                table_hbm.at[idx_v.at[0, 0, pl.ds(0, _CHUNK)]], bc.at[dst],
                semc).wait()
            pltpu.make_async_copy(
                table_hbm.at[idx_v.at[0, 0, pl.ds(0, _CHUNK)]], bd.at[dst],
                semd).wait()

        def compute(ch, rbase):
            def group_body(g, _):
                lane = lax.iota(jnp.int32, _L)
                r = lane + (rbase + g * _L)
                zero = jnp.zeros((_L,), jnp.float32)

                def dim_body(t, carry, r=r, lane=lane):
                    accs = list(carry)
                    base = t * _UNROLL
                    for u in range(_UNROLL):
                        s = u % 2
                        a1, a2, a3 = accs[3 * s], accs[3 * s + 1], accs[3 * s + 2]
                        kv = lane ^ (base + u)
                        kv2 = kv | _EMB
                        c1 = plsc.load_gather(bc, [r, kv])
                        c2 = plsc.load_gather(bc, [r, kv2])
                        d1 = plsc.load_gather(bd, [r, kv])
                        d2 = plsc.load_gather(bd, [r, kv2])
                        u_ = d1 - c1
                        ru = jnp.maximum(u_, 0.0)
                        a1 = a1 + ru * ru
                        v_ = (c2 - d2) - u_
                        rv = jnp.maximum(v_, 0.0)
                        a2 = a2 + rv * rv
                        a3 = a3 - jnp.minimum(c2, 0.0) - jnp.minimum(d2, 0.0)
                        accs[3 * s], accs[3 * s + 1], accs[3 * s + 2] = a1, a2, a3
                    return tuple(accs)

                accs = lax.fori_loop(0, _EMB // _UNROLL, dim_body, (zero,) * 6)
                a1 = accs[0] + accs[3]
                a2 = accs[1] + accs[4]
                a3 = accs[2] + accs[5]
                res = _vsqrt(a1) + _vsqrt(a2) + a3
                out_v[pl.ds(ch * _CHUNK + g * _L, _L)] = res
                return 0

            lax.fori_loop(0, groups, group_body, 0)

        start(0, 0)
        start(1, 1)

        def outer(ch, _):
            p = ch & 1

            @pl.when(p == 0)
            def _():
                wait(0)

            @pl.when(p == 1)
            def _():
                wait(1)

            compute(ch, p * _CHUNK)

            @pl.when((p == 0) & (ch + 2 < nchunk))
            def _():
                start(ch + 2, 0)

            @pl.when((p == 1) & (ch + 2 < nchunk))
            def _():
                start(ch + 2, 1)

            return 0

        lax.fori_loop(0, nchunk, outer, 0)
        pltpu.sync_copy(out_v, out_hbm.at[pl.ds(wid * pairs_per_w, pairs_per_w)])

    rowbuf = pltpu.VMEM((2 * _CHUNK, 2 * _EMB), jnp.float32)
    call = pl.kernel(
        body,
        out_type=jax.ShapeDtypeStruct((batch,), jnp.float32),
        mesh=mesh,
        scratch_types=[
            pltpu.VMEM((blocks_per_w, 2, _BLK), jnp.int32),
            rowbuf, rowbuf,
            pltpu.VMEM((pairs_per_w,), jnp.float32),
            pltpu.SemaphoreType.DMA,
            pltpu.SemaphoreType.DMA,
            pltpu.SemaphoreType.DMA,
            pltpu.SemaphoreType.DMA,
        ],
        compiler_params=pltpu.CompilerParams(
            use_tc_tiling_on_sc=True, needs_layout_passes=False,
            skip_device_barrier=True, disable_bounds_checks=True
        ),
    )
    return call(idx3, table)


def kernel(input, class_emb):
    batch = input.shape[0]
    idx3 = input.reshape(batch // _BLK, _BLK, 2).transpose(0, 2, 1)
    out = _sc_forward(idx3, class_emb, batch)
    return out.reshape(batch, 1)


# speedup vs baseline: 1.0018x; 1.0018x over previous
import functools

import jax
import jax.numpy as jnp
from jax import lax
from jax.experimental import pallas as pl
from jax.experimental.pallas import tpu as pltpu
from jax.experimental.pallas import tpu_sc as plsc

_EMB = 128
_NC = 2
_NS = 16
_NW = _NC * _NS
_L = 16
_BLK = 128

_CHUNK = 64
_UNROLL = 8


def _vsqrt(x):
    i = plsc.bitcast(x, jnp.int32)
    y = plsc.bitcast(jnp.int32(0x5F3759DF) - (i >> 1), jnp.float32)
    for _ in range(3):
        y = y * (1.5 - 0.5 * x * y * y)
    return x * y


@functools.partial(jax.jit, static_argnums=(2,))
def _sc_forward(idx3, table, batch):
    pairs_per_w = batch // _NW
    blocks_per_w = pairs_per_w // _BLK
    nchunk = pairs_per_w // _CHUNK
    groups = _CHUNK // _L
    mesh = plsc.VectorSubcoreMesh(core_axis_name="c", subcore_axis_name="s")

    def body(idx_hbm, table_hbm, out_hbm, idx_v, bc, bd, out_v,
             sc0, sd0, sc1, sd1):
        wid = lax.axis_index("s") * _NC + lax.axis_index("c")
        pltpu.sync_copy(idx_hbm.at[pl.ds(wid * blocks_per_w, blocks_per_w)],
                        idx_v)

        def start(ch, half):
            blk = ch // 2
            semc, semd = (sc0, sd0) if half == 0 else (sc1, sd1)
            dst = pl.ds(half * _CHUNK, _CHUNK)
            pltpu.async_copy(
                table_hbm.at[idx_v.at[blk, 0, pl.ds(half * _CHUNK, _CHUNK)]],
                bc.at[dst], semc)
            pltpu.async_copy(
                table_hbm.at[idx_v.at[blk, 1, pl.ds(half * _CHUNK, _CHUNK)]],
                bd.at[dst], semd)

        def wait(half):
            semc, semd = (sc0, sd0) if half == 0 else (sc1, sd1)
            dst = pl.ds(half * _CHUNK, _CHUNK)
            pltpu.make_async_copy(
                table_hbm.at[idx_v.at[0, 0, pl.ds(0, _CHUNK)]], bc.at[dst],
                semc).wait()
            pltpu.make_async_copy(
                table_hbm.at[idx_v.at[0, 0, pl.ds(0, _CHUNK)]], bd.at[dst],
                semd).wait()

        def compute(ch, rbase):
            def group_body(g, _):
                lane = lax.iota(jnp.int32, _L)
                r = lane + (rbase + g * _L)
                zero = jnp.zeros((_L,), jnp.float32)

                def dim_body(t, carry, r=r, lane=lane):
                    accs = list(carry)
                    base = t * _UNROLL
                    for u in range(_UNROLL):
                        s = u % 2
                        a1, a2, a3 = accs[3 * s], accs[3 * s + 1], accs[3 * s + 2]
                        kv = lane ^ (base + u)
                        kv2 = kv | _EMB
                        c1 = plsc.load_gather(bc, [r, kv])
                        c2 = plsc.load_gather(bc, [r, kv2])
                        d1 = plsc.load_gather(bd, [r, kv])
                        d2 = plsc.load_gather(bd, [r, kv2])
                        u_ = d1 - c1
                        ru = jnp.maximum(u_, 0.0)
                        a1 = a1 + ru * ru
                        v_ = (c2 - d2) - u_
                        rv = jnp.maximum(v_, 0.0)
                        a2 = a2 + rv * rv
                        a3 = a3 - jnp.minimum(c2, 0.0) - jnp.minimum(d2, 0.0)
                        accs[3 * s], accs[3 * s + 1], accs[3 * s + 2] = a1, a2, a3
                    return tuple(accs)

                accs = lax.fori_loop(0, _EMB // _UNROLL, dim_body, (zero,) * 6)
                a1 = accs[0] + accs[3]
                a2 = accs[1] + accs[4]
                a3 = accs[2] + accs[5]
                res = _vsqrt(a1) + _vsqrt(a2) + a3
                out_v[pl.ds(ch * _CHUNK + g * _L, _L)] = res
                return 0

            lax.fori_loop(0, groups, group_body, 0)

        start(0, 0)
        start(1, 1)

        def outer(ch, _):
            p = ch & 1

            @pl.when(p == 0)
            def _():
                wait(0)

            @pl.when(p == 1)
            def _():
                wait(1)

            compute(ch, p * _CHUNK)

            @pl.when((p == 0) & (ch + 2 < nchunk))
            def _():
                start(ch + 2, 0)

            @pl.when((p == 1) & (ch + 2 < nchunk))
            def _():
                start(ch + 2, 1)

            return 0

        lax.fori_loop(0, nchunk, outer, 0)
        pltpu.sync_copy(out_v, out_hbm.at[pl.ds(wid * pairs_per_w, pairs_per_w)])

    rowbuf = pltpu.VMEM((2 * _CHUNK, 2 * _EMB), jnp.float32)
    call = pl.kernel(
        body,
        out_type=jax.ShapeDtypeStruct((batch,), jnp.float32),
        mesh=mesh,
        scratch_types=[
            pltpu.VMEM((blocks_per_w, 2, _BLK), jnp.int32),
            rowbuf, rowbuf,
            pltpu.VMEM((pairs_per_w,), jnp.float32),
            pltpu.SemaphoreType.DMA,
            pltpu.SemaphoreType.DMA,
            pltpu.SemaphoreType.DMA,
            pltpu.SemaphoreType.DMA,
        ],
        compiler_params=pltpu.CompilerParams(
            use_tc_tiling_on_sc=True, needs_layout_passes=False
        ),
    )
    return call(idx3, table)


def kernel(input, class_emb):
    batch = input.shape[0]
    idx3 = input.reshape(batch // _BLK, _BLK, 2).transpose(0, 2, 1)
    out = _sc_forward(idx3, class_emb, batch)
    return out.reshape(batch, 1)
